# Initial kernel scaffold; baseline (speedup 1.0000x reference)
import jax
import jax.numpy as jnp
from jax.experimental import pallas as pl


def kernel(x, edge_index, W_self0, W_neigh0, b0, W_self1, W_neigh1, b1, W_self2, W_neigh2, b2):
    raise NotImplementedError("write your pallas kernel here")


# speedup vs baseline: 7.4958x; 7.4958x over previous
import functools

import jax
import jax.numpy as jnp
from jax import lax
from jax.experimental import pallas as pl
from jax.experimental.pallas import tpu as pltpu
from jax.experimental.pallas import tpu_sc as plsc

N = 10000
E = 320000
F = 128
FH = F // 2
F_OUT = 64

NC = 2
NS = 16
EPT = E // NS
CHUNK = 80
NCHUNK = EPT // CHUNK
RPT = N // NS

_mesh = plsc.VectorSubcoreMesh(core_axis_name="c", subcore_axis_name="s")


@functools.partial(
    pl.kernel,
    mesh=_mesh,
    compiler_params=pltpu.CompilerParams(use_tc_tiling_on_sc=False),
    out_type=jax.ShapeDtypeStruct((NC, NS, RPT, FH), jnp.float32),
    scratch_types=[
        pltpu.VMEM((NCHUNK, CHUNK), jnp.int32),
        pltpu.VMEM((NCHUNK, CHUNK), jnp.int32),
        pltpu.VMEM((2, CHUNK, FH), jnp.float32),
        pltpu.VMEM_SHARED((N, FH), jnp.float32),
        pltpu.SemaphoreType.DMA,
        pltpu.SemaphoreType.DMA,
    ],
)
def _segsum(h_hbm, src_hbm, dst_hbm, out_hbm,
            src_v, dst_v, rows_v, acc_sh, sem0, sem1):
    c = lax.axis_index("c")
    s = lax.axis_index("s")
    r0 = s * RPT
    hc = h_hbm.at[c]

    pltpu.sync_copy(src_hbm.at[s], src_v)
    pltpu.sync_copy(dst_hbm.at[s], dst_v)

    def zbody(r, carry):
        for k in range(FH // 16):
            rows_v[0, r, pl.ds(k * 16, 16)] = jnp.zeros((16,), jnp.float32)
        return carry

    lax.fori_loop(0, CHUNK, zbody, 0)
    for t in range(RPT // CHUNK):
        pltpu.sync_copy(rows_v.at[0], acc_sh.at[pl.ds(r0 + t * CHUNK, CHUNK)])
    _REM = RPT % CHUNK
    if _REM:
        pltpu.sync_copy(rows_v.at[0, pl.ds(0, _REM)],
                        acc_sh.at[pl.ds(r0 + (RPT // CHUNK) * CHUNK, _REM)])
    plsc.subcore_barrier()

    def start_gather(j, b):
        pltpu.async_copy(hc.at[src_v.at[j]], rows_v.at[b], sem0 if b == 0 else sem1)

    def wait_and_scatter(j, b):
        pltpu.make_async_copy(
            hc.at[src_v.at[j]], rows_v.at[b], sem0 if b == 0 else sem1
        ).wait()
        pltpu.sync_copy(rows_v.at[b], acc_sh.at[dst_v.at[j]], add=True)

    start_gather(0, 0)

    def body(i, carry):
        j = 2 * i
        start_gather(j + 1, 1)
        wait_and_scatter(j, 0)
        start_gather(j + 2, 0)
        wait_and_scatter(j + 1, 1)
        return carry

    lax.fori_loop(0, (NCHUNK - 2) // 2, body, 0)
    j = ((NCHUNK - 2) // 2) * 2
    start_gather(j + 1, 1)
    wait_and_scatter(j, 0)
    if NCHUNK % 2:
        start_gather(j + 2, 0)
        wait_and_scatter(j + 1, 1)
        wait_and_scatter(j + 2, 0)
    else:
        wait_and_scatter(j + 1, 1)

    plsc.subcore_barrier()
    pltpu.sync_copy(acc_sh.at[pl.ds(r0, RPT)], out_hbm.at[c, s])


@functools.partial(
    pl.kernel,
    mesh=_mesh,
    compiler_params=pltpu.CompilerParams(needs_layout_passes=False),
    out_type=jax.ShapeDtypeStruct((NC, NS, N), jnp.float32),
    scratch_types=[
        pltpu.VMEM((E // (NC * NS) // 16, 16), jnp.int32),
        pltpu.VMEM((N,), jnp.float32),
    ],
)
def _deg_kernel(dst_hbm, zn_hbm, out_hbm, dst_v, deg_v):
    c = lax.axis_index("c")
    s = lax.axis_index("s")
    pltpu.sync_copy(dst_hbm.at[c, s], dst_v)
    pltpu.sync_copy(zn_hbm, deg_v)
    ones = jnp.full((16,), 1.0, dtype=jnp.float32)

    def body(j, carry):
        idx = dst_v[j, :]
        plsc.addupdate_scatter(deg_v, [idx], ones)
        return carry

    lax.fori_loop(0, E // (NC * NS) // 16, body, 0)
    pltpu.sync_copy(deg_v, out_hbm.at[c, s])


def _make_tc_layer(Fout, relu, split_out):
    R = 1000
    NW = NC * NS

    def body(h0_ref, h1_ref, m0_ref, m1_ref, degt_ref,
             ws0_ref, ws1_ref, wn0_ref, wn1_ref, b_ref, o_ref):
        deg = jnp.sum(degt_ref[...], axis=1)
        recip = (1.0 / jnp.maximum(deg, 1.0))[:, None]
        acc = jnp.dot(h0_ref[...], ws0_ref[...], preferred_element_type=jnp.float32)
        acc = acc + jnp.dot(h1_ref[...], ws1_ref[...], preferred_element_type=jnp.float32)
        acc = acc + jnp.dot(m0_ref[...] * recip, wn0_ref[...],
                            preferred_element_type=jnp.float32)
        acc = acc + jnp.dot(m1_ref[...] * recip, wn1_ref[...],
                            preferred_element_type=jnp.float32)
        acc = acc + b_ref[...]
        if relu:
            acc = jnp.maximum(acc, 0.0)
        if split_out:
            o_ref[0] = acc[:, :FH]
            o_ref[1] = acc[:, FH:]
        else:
            o_ref[...] = acc

    hs = pl.BlockSpec((R, FH), lambda i: (i, 0))
    ws = pl.BlockSpec((FH, Fout), lambda i: (0, 0))
    out_shape = (jax.ShapeDtypeStruct((NC, N, FH), jnp.float32) if split_out
                 else jax.ShapeDtypeStruct((N, Fout), jnp.float32))
    out_spec = (pl.BlockSpec((NC, R, FH), lambda i: (0, i, 0)) if split_out
                else pl.BlockSpec((R, Fout), lambda i: (i, 0)))
    return pl.pallas_call(
        body,
        grid=(N // R,),
        in_specs=[hs, hs, hs, hs,
                  pl.BlockSpec((R, NW), lambda i: (i, 0)),
                  ws, ws, ws, ws,
                  pl.BlockSpec((1, Fout), lambda i: (0, 0))],
        out_specs=out_spec,
        out_shape=out_shape,
    )


_tc_layer0 = _make_tc_layer(F, True, True)
_tc_layer1 = _make_tc_layer(F, True, True)
_tc_layer2 = _make_tc_layer(F_OUT, False, False)
_TC_LAYERS = (_tc_layer0, _tc_layer1, _tc_layer2)


def kernel(x, edge_index, W_self0, W_neigh0, b0, W_self1, W_neigh1, b1,
           W_self2, W_neigh2, b2):
    src = edge_index[0].reshape(NS, NCHUNK, CHUNK)
    dst = edge_index[1].reshape(NS, NCHUNK, CHUNK)
    dstd = edge_index[1].reshape(NC, NS, E // (NC * NS) // 16, 16)
    zn = jnp.zeros((N,), jnp.float32)

    degp = _deg_kernel(dstd, zn)
    degt = degp.reshape(NC * NS, N).T

    h = jnp.stack([x[:, :FH], x[:, FH:]])
    weights = ((W_self0, W_neigh0, b0), (W_self1, W_neigh1, b1),
               (W_self2, W_neigh2, b2))
    for li in range(3):
        msg = _segsum(h, src, dst).reshape(NC, N, FH)
        wself, wneigh, b = weights[li]
        h = _TC_LAYERS[li](h[0], h[1], msg[0], msg[1], degt,
                           wself[:FH], wself[FH:], wneigh[:FH], wneigh[FH:],
                           b[None, :])
    return h


# speedup vs baseline: 9.3121x; 1.2423x over previous
import functools

import jax
import jax.numpy as jnp
from jax import lax
from jax.experimental import pallas as pl
from jax.experimental.pallas import tpu as pltpu
from jax.experimental.pallas import tpu_sc as plsc

N = 10000
E = 320000
F = 128
FH = F // 2
F_OUT = 64

NC = 2
NS = 16
EPT = E // NS
CHUNK = 125
NCHUNK = EPT // CHUNK
RING = 4
G = 40
NSUP = NCHUNK // G
RPT = N // NS

_mesh = plsc.VectorSubcoreMesh(core_axis_name="c", subcore_axis_name="s")


@functools.partial(
    pl.kernel,
    mesh=_mesh,
    compiler_params=pltpu.CompilerParams(use_tc_tiling_on_sc=False),
    out_type=jax.ShapeDtypeStruct((NC, NS, RPT, FH), jnp.float32),
    scratch_types=[
        pltpu.VMEM((2, G, 2, CHUNK), jnp.int32),
        pltpu.VMEM((RING, CHUNK, FH), jnp.float32),
        pltpu.VMEM_SHARED((N, FH), jnp.float32),
        [pltpu.SemaphoreType.DMA] * RING,
        [pltpu.SemaphoreType.DMA] * RING,
        [pltpu.SemaphoreType.DMA] * 2,
    ],
)
def _segsum(h_hbm, e_hbm, out_hbm, idx_v, rows_v, acc_sh, gsem, ssem, isem):
    c = lax.axis_index("c")
    s = lax.axis_index("s")
    r0 = s * RPT
    hc = h_hbm.at[c]

    def zbody(r, carry):
        for k in range(FH // 16):
            rows_v[0, r, pl.ds(k * 16, 16)] = jnp.zeros((16,), jnp.float32)
        return carry

    lax.fori_loop(0, CHUNK, zbody, 0)
    for t in range(RPT // CHUNK):
        pltpu.sync_copy(rows_v.at[0], acc_sh.at[pl.ds(r0 + t * CHUNK, CHUNK)])

    pltpu.sync_copy(e_hbm.at[s, 0], idx_v.at[0])
    if NSUP > 1:
        pltpu.async_copy(e_hbm.at[s, 1], idx_v.at[1], isem[1])
    plsc.subcore_barrier()

    def start_gather(p, l, r):
        pltpu.async_copy(hc.at[idx_v.at[p, l, 0]], rows_v.at[r], gsem[r])

    def wait_gather(p, l, r):
        pltpu.make_async_copy(hc.at[idx_v.at[p, l, 0]], rows_v.at[r],
                              gsem[r]).wait()

    def start_scatter(p, l, r):
        pltpu.async_copy(rows_v.at[r], acc_sh.at[idx_v.at[p, l, 1]], ssem[r],
                         add=True)

    def wait_scatter(p, l, r):
        pltpu.make_async_copy(rows_v.at[r], acc_sh.at[idx_v.at[p, l, 1]],
                              ssem[r]).wait()

    for g in range(NSUP):
        p = g % 2
        for l in range(RING - 1):
            start_gather(p, l, l)

        for r in range(RING):
            l = r
            wait_gather(p, l, r)
            start_scatter(p, l, r)
            nl = l + RING - 1
            if l >= 1:
                wait_scatter(p, l - 1, (l - 1) % RING)
            if nl < G:
                start_gather(p, nl, nl % RING)

        def body(t, carry):
            for r in range(RING):
                l = t * RING + r
                wait_gather(p, l, r)
                start_scatter(p, l, r)
                wait_scatter(p, l - 1, (r + RING - 1) % RING)
                start_gather(p, l + RING - 1, (r + RING - 1) % RING)
            return carry

        lax.fori_loop(1, G // RING - 1, body, 0)

        for r in range(RING):
            l = (G // RING - 1) * RING + r
            wait_gather(p, l, r)
            start_scatter(p, l, r)
            wait_scatter(p, l - 1, (r + RING - 1) % RING)
            nl = l + RING - 1
            if nl < G:
                start_gather(p, nl, nl % RING)

        wait_scatter(p, G - 1, (G - 1) % RING)
        if g + 2 < NSUP:
            pltpu.async_copy(e_hbm.at[s, g + 2], idx_v.at[p], isem[p])
        if g + 1 < NSUP:
            pltpu.make_async_copy(e_hbm.at[s, g + 1], idx_v.at[(g + 1) % 2],
                                  isem[(g + 1) % 2]).wait()

    plsc.subcore_barrier()
    pltpu.sync_copy(acc_sh.at[pl.ds(r0, RPT)], out_hbm.at[c, s])


@functools.partial(
    pl.kernel,
    mesh=_mesh,
    compiler_params=pltpu.CompilerParams(needs_layout_passes=False),
    out_type=jax.ShapeDtypeStruct((NC, NS, N), jnp.float32),
    scratch_types=[
        pltpu.VMEM((E // (NC * NS) // 16, 16), jnp.int32),
        pltpu.VMEM((N,), jnp.float32),
    ],
)
def _deg_kernel(dst_hbm, zn_hbm, out_hbm, dst_v, deg_v):
    c = lax.axis_index("c")
    s = lax.axis_index("s")
    pltpu.sync_copy(dst_hbm.at[c, s], dst_v)
    pltpu.sync_copy(zn_hbm, deg_v)
    ones = jnp.full((16,), 1.0, dtype=jnp.float32)

    def body(j, carry):
        idx = dst_v[j, :]
        plsc.addupdate_scatter(deg_v, [idx], ones)
        return carry

    lax.fori_loop(0, E // (NC * NS) // 16, body, 0)
    pltpu.sync_copy(deg_v, out_hbm.at[c, s])


def _make_tc_layer(Fout, relu, split_out):
    R = 1000
    NW = NC * NS

    def body(h0_ref, h1_ref, m0_ref, m1_ref, degt_ref,
             ws0_ref, ws1_ref, wn0_ref, wn1_ref, b_ref, o_ref):
        deg = jnp.sum(degt_ref[...], axis=1)
        recip = (1.0 / jnp.maximum(deg, 1.0))[:, None]
        acc = jnp.dot(h0_ref[...], ws0_ref[...], preferred_element_type=jnp.float32)
        acc = acc + jnp.dot(h1_ref[...], ws1_ref[...], preferred_element_type=jnp.float32)
        acc = acc + jnp.dot(m0_ref[...] * recip, wn0_ref[...],
                            preferred_element_type=jnp.float32)
        acc = acc + jnp.dot(m1_ref[...] * recip, wn1_ref[...],
                            preferred_element_type=jnp.float32)
        acc = acc + b_ref[...]
        if relu:
            acc = jnp.maximum(acc, 0.0)
        if split_out:
            o_ref[0] = acc[:, :FH]
            o_ref[1] = acc[:, FH:]
        else:
            o_ref[...] = acc

    hs = pl.BlockSpec((R, FH), lambda i: (i, 0))
    ws = pl.BlockSpec((FH, Fout), lambda i: (0, 0))
    out_shape = (jax.ShapeDtypeStruct((NC, N, FH), jnp.float32) if split_out
                 else jax.ShapeDtypeStruct((N, Fout), jnp.float32))
    out_spec = (pl.BlockSpec((NC, R, FH), lambda i: (0, i, 0)) if split_out
                else pl.BlockSpec((R, Fout), lambda i: (i, 0)))
    return pl.pallas_call(
        body,
        grid=(N // R,),
        in_specs=[hs, hs, hs, hs,
                  pl.BlockSpec((R, NW), lambda i: (i, 0)),
                  ws, ws, ws, ws,
                  pl.BlockSpec((1, Fout), lambda i: (0, 0))],
        out_specs=out_spec,
        out_shape=out_shape,
    )


_tc_layer0 = _make_tc_layer(F, True, True)
_tc_layer1 = _make_tc_layer(F, True, True)
_tc_layer2 = _make_tc_layer(F_OUT, False, False)
_TC_LAYERS = (_tc_layer0, _tc_layer1, _tc_layer2)


def kernel(x, edge_index, W_self0, W_neigh0, b0, W_self1, W_neigh1, b1,
           W_self2, W_neigh2, b2):
    e = jnp.concatenate(
        [edge_index[0].reshape(NS, NSUP, G, 1, CHUNK),
         edge_index[1].reshape(NS, NSUP, G, 1, CHUNK)], axis=3)
    dstd = edge_index[1].reshape(NC, NS, E // (NC * NS) // 16, 16)
    zn = jnp.zeros((N,), jnp.float32)

    degp = _deg_kernel(dstd, zn)
    degt = degp.reshape(NC * NS, N).T

    h = jnp.stack([x[:, :FH], x[:, FH:]])
    weights = ((W_self0, W_neigh0, b0), (W_self1, W_neigh1, b1),
               (W_self2, W_neigh2, b2))
    for li in range(3):
        msg = _segsum(h, e).reshape(NC, N, FH)
        wself, wneigh, b = weights[li]
        h = _TC_LAYERS[li](h[0], h[1], msg[0], msg[1], degt,
                           wself[:FH], wself[FH:], wneigh[:FH], wneigh[FH:],
                           b[None, :])
    return h


# speedup vs baseline: 11.3720x; 1.2212x over previous
import functools

import jax
import jax.numpy as jnp
from jax import lax
from jax.experimental import pallas as pl
from jax.experimental.pallas import tpu as pltpu
from jax.experimental.pallas import tpu_sc as plsc

N = 10000
E = 320000
F = 128
FH = F // 2
F_OUT = 64
DW = 16

NC = 2
NS = 16
EPT = E // NS
CHUNK = 80
NCHUNK = EPT // CHUNK
RING = 5
G = 50
NSUP = NCHUNK // G
RPT = N // NS

_mesh = plsc.VectorSubcoreMesh(core_axis_name="c", subcore_axis_name="s")


def _make_segsum(with_deg):
    out_types = [jax.ShapeDtypeStruct((NC, N, FH), jnp.float32)]
    scratch = [
        pltpu.VMEM((2, G, 2, CHUNK), jnp.int32),
        pltpu.VMEM((RING, CHUNK, FH), jnp.float32),
        pltpu.VMEM_SHARED((N, FH), jnp.float32),
        [pltpu.SemaphoreType.DMA] * RING,
        [pltpu.SemaphoreType.DMA] * RING,
        [pltpu.SemaphoreType.DMA] * 2,
    ]
    if with_deg:
        out_types.append(jax.ShapeDtypeStruct((N, DW), jnp.float32))
        scratch += [
            pltpu.VMEM((CHUNK, DW), jnp.float32),
            pltpu.VMEM_SHARED((N, DW), jnp.float32),
            pltpu.SemaphoreType.DMA,
        ]

    @functools.partial(
        pl.kernel,
        mesh=_mesh,
        compiler_params=pltpu.CompilerParams(use_tc_tiling_on_sc=False),
        out_type=tuple(out_types) if with_deg else out_types[0],
        scratch_types=scratch,
    )
    def segsum(h2_hbm, e_hbm, out_hbm, *rest):
        if with_deg:
            (deg_hbm, idx_v, rows_v, acc_sh, gsem, ssem, isem,
             ones_v, dacc_sh, dsem) = rest
        else:
            idx_v, rows_v, acc_sh, gsem, ssem, isem = rest
        c = lax.axis_index("c")
        s = lax.axis_index("s")
        r0 = s * RPT

        def zbody(r, carry):
            for k in range(FH // 16):
                rows_v[0, r, pl.ds(k * 16, 16)] = jnp.zeros((16,), jnp.float32)
            return carry

        lax.fori_loop(0, CHUNK, zbody, 0)
        for t in range(RPT // CHUNK):
            pltpu.sync_copy(rows_v.at[0], acc_sh.at[pl.ds(r0 + t * CHUNK, CHUNK)])
        _REM = RPT % CHUNK
        if _REM:
            pltpu.sync_copy(rows_v.at[0, pl.ds(0, _REM)],
                            acc_sh.at[pl.ds(r0 + (RPT // CHUNK) * CHUNK, _REM)])

        if with_deg:
            def obody(r, carry):
                ones_v[r, pl.ds(0, DW)] = jnp.zeros((DW,), jnp.float32)
                return carry

            lax.fori_loop(0, CHUNK, obody, 0)
            for t in range(RPT // CHUNK):
                pltpu.sync_copy(ones_v, dacc_sh.at[pl.ds(r0 + t * CHUNK, CHUNK)])
            if _REM:
                pltpu.sync_copy(ones_v.at[pl.ds(0, _REM)],
                                dacc_sh.at[pl.ds(r0 + (RPT // CHUNK) * CHUNK, _REM)])

            def o1body(r, carry):
                ones_v[r, pl.ds(0, DW)] = jnp.full((DW,), 1.0, jnp.float32)
                return carry

            lax.fori_loop(0, CHUNK, o1body, 0)

        cc = lax.broadcast(c, (16,))

        def transform_super(p):
            def tbody(l, carry):
                for k in range(CHUNK // 16):
                    v = idx_v[p, l, 0, pl.ds(k * 16, 16)]
                    idx_v[p, l, 0, pl.ds(k * 16, 16)] = v + v + cc
                return carry

            lax.fori_loop(0, G, tbody, 0)

        pltpu.sync_copy(e_hbm.at[s, 0], idx_v.at[0])
        if NSUP > 1:
            pltpu.async_copy(e_hbm.at[s, 1], idx_v.at[1], isem[1])
        transform_super(0)
        plsc.subcore_barrier()

        def start_gather(p, l, r):
            pltpu.async_copy(h2_hbm.at[idx_v.at[p, l, 0]], rows_v.at[r], gsem[r])

        def wait_gather(p, l, r):
            pltpu.make_async_copy(h2_hbm.at[idx_v.at[p, l, 0]], rows_v.at[r],
                                  gsem[r]).wait()

        def start_scatter(p, l, r):
            pltpu.async_copy(rows_v.at[r], acc_sh.at[idx_v.at[p, l, 1]], ssem[r],
                             add=True)
            if with_deg:
                pltpu.async_copy(ones_v, dacc_sh.at[idx_v.at[p, l, 1]], dsem,
                                 add=True)

        def wait_scatter(p, l, r):
            pltpu.make_async_copy(rows_v.at[r], acc_sh.at[idx_v.at[p, l, 1]],
                                  ssem[r]).wait()
            if with_deg:
                pltpu.make_async_copy(ones_v, dacc_sh.at[idx_v.at[p, l, 1]],
                                      dsem).wait()

        for g in range(NSUP):
            p = g % 2
            for l in range(RING - 1):
                start_gather(p, l, l)

            for r in range(RING):
                l = r
                wait_gather(p, l, r)
                start_scatter(p, l, r)
                if l >= 1:
                    wait_scatter(p, l - 1, (l - 1) % RING)
                nl = l + RING - 1
                if nl < G:
                    start_gather(p, nl, nl % RING)

            def body(t, carry):
                for r in range(RING):
                    l = t * RING + r
                    wait_gather(p, l, r)
                    start_scatter(p, l, r)
                    wait_scatter(p, l - 1, (r + RING - 1) % RING)
                    start_gather(p, l + RING - 1, (r + RING - 1) % RING)
                return carry

            lax.fori_loop(1, G // RING - 1, body, 0)

            for r in range(RING):
                l = (G // RING - 1) * RING + r
                wait_gather(p, l, r)
                start_scatter(p, l, r)
                wait_scatter(p, l - 1, (r + RING - 1) % RING)
                nl = l + RING - 1
                if nl < G:
                    start_gather(p, nl, nl % RING)

            wait_scatter(p, G - 1, (G - 1) % RING)
            if g + 2 < NSUP:
                pltpu.async_copy(e_hbm.at[s, g + 2], idx_v.at[p], isem[p])
            if g + 1 < NSUP:
                pltpu.make_async_copy(e_hbm.at[s, g + 1], idx_v.at[(g + 1) % 2],
                                      isem[(g + 1) % 2]).wait()
                transform_super((g + 1) % 2)

        plsc.subcore_barrier()
        pltpu.sync_copy(acc_sh.at[pl.ds(r0, RPT)], out_hbm.at[c, pl.ds(r0, RPT)])
        if with_deg:
            @pl.when(c == 0)
            def _():
                pltpu.sync_copy(dacc_sh.at[pl.ds(r0, RPT)],
                                deg_hbm.at[pl.ds(r0, RPT)])

    return segsum


_segsum = _make_segsum(False)
_segsum_deg = _make_segsum(True)


def _make_tc_layer(Fout, relu):
    R = 1000

    def body(h_ref, m0_ref, m1_ref, deg_ref, ws_ref, wn0_ref, wn1_ref, b_ref,
             o_ref):
        recip = 1.0 / jnp.maximum(deg_ref[:, 0:1], 1.0)
        acc = jnp.dot(h_ref[...], ws_ref[...], preferred_element_type=jnp.float32)
        acc = acc + jnp.dot(m0_ref[0] * recip, wn0_ref[...],
                            preferred_element_type=jnp.float32)
        acc = acc + jnp.dot(m1_ref[0] * recip, wn1_ref[...],
                            preferred_element_type=jnp.float32)
        acc = acc + b_ref[...]
        if relu:
            acc = jnp.maximum(acc, 0.0)
        o_ref[...] = acc

    ms = pl.BlockSpec((1, R, FH), lambda i: (0, i, 0))
    ms1 = pl.BlockSpec((1, R, FH), lambda i: (1, i, 0))
    wn = pl.BlockSpec((FH, Fout), lambda i: (0, 0))
    return pl.pallas_call(
        body,
        grid=(N // R,),
        in_specs=[
            pl.BlockSpec((R, F), lambda i: (i, 0)),
            ms, ms1,
            pl.BlockSpec((R, DW), lambda i: (i, 0)),
            pl.BlockSpec((F, Fout), lambda i: (0, 0)),
            wn, wn,
            pl.BlockSpec((1, Fout), lambda i: (0, 0)),
        ],
        out_specs=pl.BlockSpec((R, Fout), lambda i: (i, 0)),
        out_shape=jax.ShapeDtypeStruct((N, Fout), jnp.float32),
    )


_TC_LAYERS = (_make_tc_layer(F, True), _make_tc_layer(F, True),
              _make_tc_layer(F_OUT, False))


def kernel(x, edge_index, W_self0, W_neigh0, b0, W_self1, W_neigh1, b1,
           W_self2, W_neigh2, b2):
    e = jnp.concatenate(
        [edge_index[0].reshape(NS, NSUP, G, 1, CHUNK),
         edge_index[1].reshape(NS, NSUP, G, 1, CHUNK)], axis=3)

    h = x
    weights = ((W_self0, W_neigh0, b0), (W_self1, W_neigh1, b1),
               (W_self2, W_neigh2, b2))
    msg, deg = _segsum_deg(x.reshape(2 * N, FH), e)
    for li in range(3):
        wself, wneigh, b = weights[li]
        h = _TC_LAYERS[li](h, msg, msg, deg, wself,
                           wneigh[:FH], wneigh[FH:], b[None, :])
        if li < 2:
            msg = _segsum(h.reshape(2 * N, FH), e)
    return h


# speedup vs baseline: 12.3441x; 1.0855x over previous
import functools

import jax
import jax.numpy as jnp
from jax import lax
from jax.experimental import pallas as pl
from jax.experimental.pallas import tpu as pltpu
from jax.experimental.pallas import tpu_sc as plsc

N = 10000
E = 320000
F = 128
FH = F // 2
F_OUT = 64
DW = 16

NC = 2
NS = 16
EPT = E // NS
CHUNK = 80
NCHUNK = EPT // CHUNK
RING = 5
G = 50
NSUP = NCHUNK // G
RPT = N // NS

_mesh = plsc.VectorSubcoreMesh(core_axis_name="c", subcore_axis_name="s")


def _make_segsum(with_deg):
    out_types = [jax.ShapeDtypeStruct((NC, N, FH), jnp.float32)]
    scratch = [
        pltpu.VMEM((2, G, CHUNK), jnp.int32),
        pltpu.VMEM((2, G, CHUNK), jnp.int32),
        pltpu.VMEM((RING, CHUNK, FH), jnp.float32),
        pltpu.VMEM_SHARED((N, FH), jnp.float32),
        [pltpu.SemaphoreType.DMA] * RING,
        [pltpu.SemaphoreType.DMA] * RING,
        [pltpu.SemaphoreType.DMA] * 2,
    ]
    if with_deg:
        out_types.append(jax.ShapeDtypeStruct((N, DW), jnp.float32))
        scratch += [
            pltpu.VMEM((CHUNK, DW), jnp.float32),
            pltpu.VMEM_SHARED((N, DW), jnp.float32),
            pltpu.SemaphoreType.DMA,
        ]

    @functools.partial(
        pl.kernel,
        mesh=_mesh,
        compiler_params=pltpu.CompilerParams(use_tc_tiling_on_sc=False),
        out_type=tuple(out_types) if with_deg else out_types[0],
        scratch_types=scratch,
    )
    def segsum(h2, e_hbm, out_hbm, *rest):
        if with_deg:
            (deg_hbm, src_v, dst_v, rows_v, acc_sh, gsem, ssem, isem,
             ones_v, dacc_sh, dsem) = rest
        else:
            src_v, dst_v, rows_v, acc_sh, gsem, ssem, isem = rest
        c = lax.axis_index("c")
        s = lax.axis_index("s")
        r0 = s * RPT

        def zbody(r, carry):
            for k in range(FH // 16):
                rows_v[0, r, pl.ds(k * 16, 16)] = jnp.zeros((16,), jnp.float32)
            return carry

        lax.fori_loop(0, CHUNK, zbody, 0)
        for t in range(RPT // CHUNK):
            pltpu.sync_copy(rows_v.at[0], acc_sh.at[pl.ds(r0 + t * CHUNK, CHUNK)])
        _REM = RPT % CHUNK
        if _REM:
            pltpu.sync_copy(rows_v.at[0, pl.ds(0, _REM)],
                            acc_sh.at[pl.ds(r0 + (RPT // CHUNK) * CHUNK, _REM)])

        if with_deg:
            def obody(r, carry):
                ones_v[r, pl.ds(0, DW)] = jnp.zeros((DW,), jnp.float32)
                return carry

            lax.fori_loop(0, CHUNK, obody, 0)
            for t in range(RPT // CHUNK):
                pltpu.sync_copy(ones_v, dacc_sh.at[pl.ds(r0 + t * CHUNK, CHUNK)])
            if _REM:
                pltpu.sync_copy(ones_v.at[pl.ds(0, _REM)],
                                dacc_sh.at[pl.ds(r0 + (RPT // CHUNK) * CHUNK, _REM)])

            def o1body(r, carry):
                ones_v[r, pl.ds(0, DW)] = jnp.full((DW,), 1.0, jnp.float32)
                return carry

            lax.fori_loop(0, CHUNK, o1body, 0)

        cc = lax.broadcast(c, (16,))

        def transform_super(p):
            def tbody(l, carry):
                for k in range(CHUNK // 16):
                    v = src_v[p, l, pl.ds(k * 16, 16)]
                    src_v[p, l, pl.ds(k * 16, 16)] = v + v + cc
                return carry

            lax.fori_loop(0, G, tbody, 0)

        pltpu.sync_copy(e_hbm.at[0, s, 0], src_v.at[0])
        pltpu.sync_copy(e_hbm.at[1, s, 0], dst_v.at[0])
        if NSUP > 1:
            pltpu.async_copy(e_hbm.at[0, s, 1], src_v.at[1], isem[1])
            pltpu.async_copy(e_hbm.at[1, s, 1], dst_v.at[1], isem[1])
        transform_super(0)
        plsc.subcore_barrier()

        def start_gather(p, l, r):
            pltpu.async_copy(h2.at[src_v.at[p, l]], rows_v.at[r], gsem[r])

        def wait_gather(p, l, r):
            pltpu.make_async_copy(h2.at[src_v.at[p, l]], rows_v.at[r],
                                  gsem[r]).wait()

        def start_scatter(p, l, r):
            pltpu.async_copy(rows_v.at[r], acc_sh.at[dst_v.at[p, l]], ssem[r],
                             add=True)
            if with_deg:
                pltpu.async_copy(ones_v, dacc_sh.at[dst_v.at[p, l]], dsem,
                                 add=True)

        def wait_scatter(p, l, r):
            pltpu.make_async_copy(rows_v.at[r], acc_sh.at[dst_v.at[p, l]],
                                  ssem[r]).wait()
            if with_deg:
                pltpu.make_async_copy(ones_v, dacc_sh.at[dst_v.at[p, l]],
                                      dsem).wait()

        for g in range(NSUP):
            p = g % 2
            for l in range(RING - 1):
                start_gather(p, l, l)

            for r in range(RING):
                l = r
                wait_gather(p, l, r)
                start_scatter(p, l, r)
                if l >= 1:
                    wait_scatter(p, l - 1, (l - 1) % RING)
                nl = l + RING - 1
                if nl < G:
                    start_gather(p, nl, nl % RING)

            def body(t, carry):
                for r in range(RING):
                    l = t * RING + r
                    wait_gather(p, l, r)
                    start_scatter(p, l, r)
                    wait_scatter(p, l - 1, (r + RING - 1) % RING)
                    start_gather(p, l + RING - 1, (r + RING - 1) % RING)
                return carry

            lax.fori_loop(1, G // RING - 1, body, 0)

            for r in range(RING):
                l = (G // RING - 1) * RING + r
                wait_gather(p, l, r)
                start_scatter(p, l, r)
                wait_scatter(p, l - 1, (r + RING - 1) % RING)
                nl = l + RING - 1
                if nl < G:
                    start_gather(p, nl, nl % RING)

            wait_scatter(p, G - 1, (G - 1) % RING)
            if g + 2 < NSUP:
                pltpu.async_copy(e_hbm.at[0, s, g + 2], src_v.at[p], isem[p])
                pltpu.async_copy(e_hbm.at[1, s, g + 2], dst_v.at[p], isem[p])
            if g + 1 < NSUP:
                q = (g + 1) % 2
                pltpu.make_async_copy(e_hbm.at[0, s, g + 1], src_v.at[q],
                                      isem[q]).wait()
                pltpu.make_async_copy(e_hbm.at[1, s, g + 1], dst_v.at[q],
                                      isem[q]).wait()
                transform_super(q)

        plsc.subcore_barrier()
        pltpu.sync_copy(acc_sh.at[pl.ds(r0, RPT)], out_hbm.at[c, pl.ds(r0, RPT)])
        if with_deg:
            @pl.when(c == 0)
            def _():
                pltpu.sync_copy(dacc_sh.at[pl.ds(r0, RPT)],
                                deg_hbm.at[pl.ds(r0, RPT)])

    return segsum


_segsum = _make_segsum(False)
_segsum_deg = _make_segsum(True)


def _make_tc_layer(Fout, relu):
    R = 1000

    def body(h_ref, m0_ref, m1_ref, deg_ref, ws_ref, wn0_ref, wn1_ref, b_ref,
             o_ref):
        recip = 1.0 / jnp.maximum(deg_ref[:, 0:1], 1.0)
        acc = jnp.dot(h_ref[...], ws_ref[...], preferred_element_type=jnp.float32)
        acc = acc + jnp.dot(m0_ref[0] * recip, wn0_ref[...],
                            preferred_element_type=jnp.float32)
        acc = acc + jnp.dot(m1_ref[0] * recip, wn1_ref[...],
                            preferred_element_type=jnp.float32)
        acc = acc + b_ref[...]
        if relu:
            acc = jnp.maximum(acc, 0.0)
        o_ref[...] = acc

    ms = pl.BlockSpec((1, R, FH), lambda i: (0, i, 0))
    ms1 = pl.BlockSpec((1, R, FH), lambda i: (1, i, 0))
    wn = pl.BlockSpec((FH, Fout), lambda i: (0, 0))
    return pl.pallas_call(
        body,
        grid=(N // R,),
        in_specs=[
            pl.BlockSpec((R, F), lambda i: (i, 0)),
            ms, ms1,
            pl.BlockSpec((R, DW), lambda i: (i, 0)),
            pl.BlockSpec((F, Fout), lambda i: (0, 0)),
            wn, wn,
            pl.BlockSpec((1, Fout), lambda i: (0, 0)),
        ],
        out_specs=pl.BlockSpec((R, Fout), lambda i: (i, 0)),
        out_shape=jax.ShapeDtypeStruct((N, Fout), jnp.float32),
    )


_TC_LAYERS = (_make_tc_layer(F, True), _make_tc_layer(F, True),
              _make_tc_layer(F_OUT, False))


def kernel(x, edge_index, W_self0, W_neigh0, b0, W_self1, W_neigh1, b1,
           W_self2, W_neigh2, b2):
    e = edge_index.reshape(2, NS, NSUP, G, CHUNK)

    h = x
    weights = ((W_self0, W_neigh0, b0), (W_self1, W_neigh1, b1),
               (W_self2, W_neigh2, b2))
    msg, deg = _segsum_deg(x.reshape(2 * N, FH), e)
    for li in range(3):
        wself, wneigh, b = weights[li]
        h = _TC_LAYERS[li](h, msg, msg, deg, wself,
                           wneigh[:FH], wneigh[FH:], b[None, :])
        if li < 2:
            msg = _segsum(h.reshape(2 * N, FH), e)
    return h
